# Initial kernel scaffold; baseline (speedup 1.0000x reference)
import jax
import jax.numpy as jnp
from jax.experimental import pallas as pl


def kernel(x, edge_index, edge_attr, colour_idx, W_enc, b_enc, W_edge, b_edge, eps, W1, b1, W2, b2, colour_vec, W_head, b_head):
    raise NotImplementedError("write your pallas kernel here")


# speedup vs baseline: 13.0864x; 13.0864x over previous
import functools

import jax
import jax.numpy as jnp
from jax import lax
from jax.experimental import pallas as pl
from jax.experimental.pallas import tpu as pltpu
from jax.experimental.pallas import tpu_sc as plsc

N = 10000
E = 320000
D = 128
D_IN = 128
D_EDGE = 16
S = 5
L = 3
D_OUT = 10

NC = 2
NS = 16
NW = NC * NS
EW = E // NW
BLK = 80
NB = EW // BLK
NP = 10240
STRIPE = NP // NS
C16 = D // 16


def _msg_body(hc, e, src, dst, out, srcb, dstb, ebuf, rows, outb, msg_sh, sem):
    cid = lax.axis_index("c")
    sid = lax.axis_index("s")
    wid = cid * NS + sid

    def zrow(r, carry):
        for c in range(C16):
            outb[r, pl.ds(c * 16, 16)] = jnp.zeros((16,), jnp.float32)
        return carry

    lax.fori_loop(0, BLK, zrow, 0)
    row0 = sid * STRIPE
    for k in range(STRIPE // BLK):
        pltpu.sync_copy(outb, msg_sh.at[pl.ds(row0 + k * BLK, BLK)])

    plsc.subcore_barrier()

    ebase = wid * EW

    def block(j, carry):
        pltpu.sync_copy(src.at[wid, pl.ds(j, 1)], srcb)
        pltpu.sync_copy(dst.at[wid, pl.ds(j, 1)], dstb)
        pltpu.sync_copy(e.at[pl.ds(ebase + j * BLK, BLK)], ebuf)
        pltpu.async_copy(hc.at[srcb.at[0]], rows, sem).wait()

        def edge(i, c2):
            for c in range(C16):
                sl = pl.ds(c * 16, 16)
                outb[i, sl] = jnp.maximum(rows[i, sl] + ebuf[i, sl], 0.0)
            return c2

        lax.fori_loop(0, BLK, edge, 0)
        pltpu.sync_copy(outb, msg_sh.at[dstb.at[0]], add=True)
        return carry

    lax.fori_loop(0, NB, block, 0)
    plsc.subcore_barrier()
    pltpu.sync_copy(msg_sh.at[pl.ds(row0, STRIPE)],
                    out.at[cid, pl.ds(row0, STRIPE)])


_msg_call = functools.partial(
    pl.kernel,
    out_type=jax.ShapeDtypeStruct((NC, NP, D), jnp.float32),
    mesh=plsc.VectorSubcoreMesh(core_axis_name="c", subcore_axis_name="s"),
    scratch_types=[
        pltpu.VMEM((1, BLK), jnp.int32),
        pltpu.VMEM((1, BLK), jnp.int32),
        pltpu.VMEM((BLK, D), jnp.float32),
        pltpu.VMEM((BLK, D), jnp.float32),
        pltpu.VMEM((BLK, D), jnp.float32),
        pltpu.VMEM_SHARED((NP, D), jnp.float32),
        pltpu.SemaphoreType.DMA,
    ],
)(_msg_body)


def _enc_body(x_ref, w_ref, b_ref, o_ref):
    o_ref[...] = jnp.maximum(
        jnp.dot(x_ref[...], w_ref[...], preferred_element_type=jnp.float32)
        + b_ref[...], 0.0)


def _encoder(x, W, b):
    R = 1000
    return pl.pallas_call(
        _enc_body,
        grid=(N // R,),
        in_specs=[pl.BlockSpec((R, D_IN), lambda i: (i, 0)),
                  pl.BlockSpec((D_IN, D), lambda i: (0, 0)),
                  pl.BlockSpec((1, D), lambda i: (0, 0))],
        out_specs=pl.BlockSpec((R, D), lambda i: (i, 0)),
        out_shape=jax.ShapeDtypeStruct((N, D), jnp.float32),
    )(x, W, b.reshape(1, D))


def _edge_mlp(ea, W, b):
    R = 4000
    return pl.pallas_call(
        _enc_body,
        grid=(E // R,),
        in_specs=[pl.BlockSpec((R, D_EDGE), lambda i: (i, 0)),
                  pl.BlockSpec((D_EDGE, D), lambda i: (0, 0)),
                  pl.BlockSpec((1, D), lambda i: (0, 0))],
        out_specs=pl.BlockSpec((R, D), lambda i: (i, 0)),
        out_shape=jax.ShapeDtypeStruct((E, D), jnp.float32),
    )(ea, W, b.reshape(1, D))


def _mlp_body(hs_ref, hc_ref, m0_ref, m1_ref, sc_ref, w1_ref, b1_ref,
              w2_ref, b2_ref, o_ref):
    pre = hc_ref[...] * sc_ref[...] + m0_ref[...] + m1_ref[...]
    t = jnp.maximum(
        jnp.dot(pre, w1_ref[...], preferred_element_type=jnp.float32)
        + b1_ref[...], 0.0)
    u = jnp.dot(t, w2_ref[...], preferred_element_type=jnp.float32) + b2_ref[...]
    o_ref[...] = hs_ref[...] + jnp.maximum(u, 0.0)


def _mlp(hs, hc, m0, m1, scale, W1, b1, W2, b2):
    R = 1000
    full = pl.BlockSpec((D, D), lambda i: (0, 0))
    row = pl.BlockSpec((1, D), lambda i: (0, 0))
    blk = pl.BlockSpec((R, D), lambda i: (i, 0))
    return pl.pallas_call(
        _mlp_body,
        grid=(N // R,),
        in_specs=[blk, blk, blk, blk, row, full, row, full, row],
        out_specs=blk,
        out_shape=jax.ShapeDtypeStruct((N, D), jnp.float32),
    )(hs, hc, m0, m1, scale, W1, b1.reshape(1, D), W2, b2.reshape(1, D))


def _read_body(h0, h1, h2, h3, h4, wh, bh, o_ref):
    acc = h0[...] + h1[...] + h2[...] + h3[...] + h4[...]
    pooled = jnp.sum(acc, axis=0, keepdims=True) * (1.0 / (S * N))
    o_ref[...] = jnp.dot(pooled, wh[...],
                         preferred_element_type=jnp.float32) + bh[...]


def _readout(hs, W_head, b_head):
    nd = pl.BlockSpec((N, D), lambda: (0, 0))
    out = pl.pallas_call(
        _read_body,
        in_specs=[nd, nd, nd, nd, nd,
                  pl.BlockSpec((D, D_OUT), lambda: (0, 0)),
                  pl.BlockSpec((1, D_OUT), lambda: (0, 0))],
        out_specs=pl.BlockSpec((1, D_OUT), lambda: (0, 0)),
        out_shape=jax.ShapeDtypeStruct((1, D_OUT), jnp.float32),
    )(*hs, W_head, b_head.reshape(1, D_OUT))
    return out[0]


def kernel(x, edge_index, edge_attr, colour_idx, W_enc, b_enc, W_edge,
           b_edge, eps, W1, b1, W2, b2, colour_vec, W_head, b_head):
    src = edge_index[0].reshape(NW, NB, BLK)
    dst = edge_index[1].reshape(NW, NB, BLK)
    h = _encoder(x, W_enc, b_enc)
    hs = [h] * S
    for l in range(L):
        e = _edge_mlp(edge_attr, W_edge[l], b_edge[l])
        scale = jnp.broadcast_to(1.0 + eps[l], (1, D))
        new_hs = []
        for s in range(S):
            hc = hs[s].at[colour_idx[s]].add(colour_vec[l])
            msg = _msg_call(hc, e, src, dst)
            new_hs.append(_mlp(hs[s], hc, msg[0], msg[1], scale,
                               W1[l], b1[l], W2[l], b2[l]))
        hs = new_hs
    return _readout(hs, W_head, b_head)
